# baseline (speedup 1.0000x reference)
import functools

import jax
import jax.numpy as jnp
from jax import lax
from jax.experimental import pallas as pl
from jax.experimental.pallas import tpu as pltpu
from jax.experimental.pallas import tpu_sc as plsc

N = 10000
E = 160000
NP = 10240
CH = 112
EP = 161280
NSUB = 16
ROWS_PER_SUB = NP // NSUB
GAH = 1
NBUF = GAH + 2
IDXB = GAH + 3

_MESH = plsc.VectorSubcoreMesh(core_axis_name="c", subcore_axis_name="s")


def _make_prop(dh, feat_split, dup_table=False):
    nc = (EP if feat_split else EP // 2) // NSUB // CH
    xform = feat_split or dup_table
    row_shape = (CH, dh) if dh > 1 else (CH,)

    @functools.partial(
        pl.kernel,
        out_type=jax.ShapeDtypeStruct((2, NP, dh) if dh > 1 else (2, NP),
                                      jnp.float32),
        mesh=_MESH,
        scratch_types=(
            [pltpu.VMEM((2, CH), jnp.int32) for _ in range(IDXB)]
            + [pltpu.VMEM(row_shape, jnp.float32) for _ in range(NBUF)]
            + [pltpu.VMEM_SHARED((NP, dh) if dh > 1 else (NP,),
                                 jnp.float32)]
            + [pltpu.SemaphoreType.DMA for _ in range(2 * NBUF + IDXB)]),
    )
    def prop(table, sd_hbm, out, *scr):
        idx_b = scr[:IDXB]
        bufs = scr[IDXB:IDXB + NBUF]
        acc = scr[IDXB + NBUF]
        o = IDXB + NBUF + 1
        gsems = scr[o:o + NBUF]
        ssems = scr[o + NBUF:o + 2 * NBUF]
        isems = scr[o + 2 * NBUF:o + 2 * NBUF + IDXB]
        c = lax.axis_index("c")
        s = lax.axis_index("s")

        z = jnp.zeros((16,), jnp.float32)
        if dh > 1:
            def zrow(i, _):
                for j in range(dh // 16):
                    bufs[0][i, pl.ds(j * 16, 16)] = z
                return 0

            lax.fori_loop(0, CH, zrow, 0, unroll=4)
        else:
            for j in range(CH // 16):
                bufs[0][pl.ds(j * 16, 16)] = z
        base_r = s * ROWS_PER_SUB
        zfull, zrem = divmod(ROWS_PER_SUB, CH)
        for kz in range(zfull):
            pltpu.sync_copy(bufs[0], acc.at[pl.ds(base_r + kz * CH, CH)])
        if zrem:
            pltpu.sync_copy(bufs[0].at[pl.ds(0, zrem)],
                            acc.at[pl.ds(base_r + zfull * CH, zrem)])
        plsc.subcore_barrier()

        off = c * NP

        def fire_idx(k):
            ib = k % IDXB
            if feat_split:
                g = s * nc + k
            else:
                g = (k * 2 + c) * NSUB + s
            return pltpu.async_copy(sd_hbm.at[g], idx_b[ib], isems[ib])

        def prep_gather(k):
            ib = k % IDXB
            idx_d[k].wait()
            if xform:
                for j in range(CH // 16):
                    idx_b[ib][0, pl.ds(j * 16, 16)] = (
                        idx_b[ib][0, pl.ds(j * 16, 16)] + off)
            return pltpu.async_copy(table.at[idx_b[ib].at[0]],
                                    bufs[k % NBUF], gsems[k % NBUF])

        idx_d = [None] * nc
        gdl = [None] * nc
        sd = [None] * nc
        for k in range(min(GAH + 1, nc)):
            idx_d[k] = fire_idx(k)
        for k in range(min(GAH, nc)):
            gdl[k] = prep_gather(k)
        for k in range(nc):
            if k >= 2:
                sd[k - 2].wait()
            if k + GAH + 1 < nc:
                idx_d[k + GAH + 1] = fire_idx(k + GAH + 1)
            if k + GAH < nc:
                gdl[k + GAH] = prep_gather(k + GAH)
            gdl[k].wait()
            sd[k] = pltpu.async_copy(bufs[k % NBUF],
                                     acc.at[idx_b[k % IDXB].at[1]],
                                     ssems[k % NBUF], add=True)
        for k in range(max(0, nc - 2), nc):
            sd[k].wait()
        plsc.subcore_barrier()
        pltpu.sync_copy(
            acc.at[pl.ds(base_r, ROWS_PER_SUB)],
            out.at[c, pl.ds(base_r, ROWS_PER_SUB)],
        )

    return prop


def _make_count():
    nc = EP // 2 // NSUB // CH

    @functools.partial(
        pl.kernel,
        out_type=jax.ShapeDtypeStruct((2, NP), jnp.float32),
        mesh=_MESH,
        scratch_types=[
            pltpu.VMEM((nc, CH), jnp.int32),
            pltpu.VMEM((CH,), jnp.float32),
            pltpu.VMEM_SHARED((NP,), jnp.float32),
            pltpu.SemaphoreType.DMA,
        ],
    )
    def count(dstp, out, dst_loc, ones_v, acc, sem):
        c = lax.axis_index("c")
        s = lax.axis_index("s")
        pltpu.sync_copy(dstp.at[c, s], dst_loc)
        z = jnp.zeros((16,), jnp.float32)
        for j in range(CH // 16):
            ones_v[pl.ds(j * 16, 16)] = z
        base_r = s * ROWS_PER_SUB
        zfull, zrem = divmod(ROWS_PER_SUB, CH)
        for kz in range(zfull):
            pltpu.sync_copy(ones_v, acc.at[pl.ds(base_r + kz * CH, CH)])
        if zrem:
            pltpu.sync_copy(ones_v.at[pl.ds(0, zrem)],
                            acc.at[pl.ds(base_r + zfull * CH, zrem)])
        one = jnp.ones((16,), jnp.float32)
        for j in range(CH // 16):
            ones_v[pl.ds(j * 16, 16)] = one
        plsc.subcore_barrier()
        sds = []
        for k in range(nc):
            sds.append(pltpu.async_copy(ones_v, acc.at[dst_loc.at[k]], sem,
                                        add=True))
        for d in sds:
            d.wait()
        plsc.subcore_barrier()
        pltpu.sync_copy(
            acc.at[pl.ds(base_r, ROWS_PER_SUB)],
            out.at[c, pl.ds(base_r, ROWS_PER_SUB)],
        )

    return count


_R = 512


def _tc_a0_body(x_ref, w1_ref, h_ref):
    h_ref[...] = jnp.dot(x_ref[...], w1_ref[...],
                         preferred_element_type=jnp.float32)


def _tc_a1_body(h_ref, d0_ref, d1_ref, hs_ref, dinv_ref):
    dinv = lax.rsqrt(d0_ref[...] + d1_ref[...] + 1.0)
    hs = h_ref[...] * dinv
    hs_ref[0] = hs[:, :128]
    hs_ref[1] = hs[:, 128:]
    dinv_ref[...] = dinv


def _tc_c_body(s_ref, hs_ref, dinv_ref, b1_ref, w2_ref, out_ref):
    dinv = dinv_ref[...]
    b1 = b1_ref[...]
    w2 = w2_ref[...]
    t0 = jax.nn.relu(dinv * (s_ref[0] + hs_ref[0]) + b1[0][None, :])
    t1 = jax.nn.relu(dinv * (s_ref[1] + hs_ref[1]) + b1[1][None, :])
    h2 = (jnp.dot(t0, w2[:128], preferred_element_type=jnp.float32)
          + jnp.dot(t1, w2[128:], preferred_element_type=jnp.float32))
    hs2 = h2 * dinv
    out_ref[0] = hs2
    out_ref[1] = hs2


def _tc_e_body(s_ref, hs_ref, dinv_ref, b2_ref, w3_ref, out_ref):
    dinv = dinv_ref[...]
    t = jax.nn.relu(dinv * (s_ref[0] + s_ref[1] + hs_ref[0]) + b2_ref[...])
    v = jnp.dot(t, w3_ref[...], preferred_element_type=jnp.float32)
    out_ref[...] = v * dinv


def _tc_g_body(s0_ref, s1_ref, vs_ref, dinv_ref, b3_ref, fcw_ref, fcb_ref,
               out_ref):
    h3 = (dinv_ref[...] * (s0_ref[...] + s1_ref[...] + vs_ref[...])
          + b3_ref[0, 0])
    out_ref[...] = h3 * fcw_ref[...] + fcb_ref[...]


def _row_spec(cols):
    return pl.BlockSpec((_R, cols), lambda i: (i, 0))


def _whole_spec(shape):
    return pl.BlockSpec(shape, lambda i: tuple(0 for _ in shape))


def _half_spec(cols):
    return pl.BlockSpec((2, _R, cols), lambda i: (0, i, 0))


def kernel(x, edge_index, W1, b1, W2, b2, W3, b3, fc_W, fc_b):
    grid = (NP // _R,)

    src = edge_index[0].astype(jnp.int32)
    dst = edge_index[1].astype(jnp.int32)
    pad_dst = N + (jnp.arange(EP - E, dtype=jnp.int32) % (NP - N))
    srcp = jnp.concatenate([src, jnp.full((EP - E,), N, jnp.int32)])
    dstp = jnp.concatenate([dst, pad_dst])
    sd_arr = jnp.stack([srcp.reshape(EP // CH, CH),
                        dstp.reshape(EP // CH, CH)], axis=1)
    nc_e = EP // 2 // NSUB // CH
    dst_e = dstp.reshape(2, NSUB, nc_e, CH)

    xp = jnp.pad(x, ((0, NP - N), (0, 0)))

    count1d = _make_count()
    prop_l1 = _make_prop(128, feat_split=True)
    prop_l2 = _make_prop(128, feat_split=False, dup_table=True)
    prop_l3 = _make_prop(1, feat_split=False)

    degp = count1d(dst_e)
    d0 = degp[0].reshape(NP, 1)
    d1 = degp[1].reshape(NP, 1)

    h1 = pl.pallas_call(
        _tc_a0_body,
        grid=grid,
        in_specs=[_row_spec(256), _whole_spec((256, 256))],
        out_specs=_row_spec(256),
        out_shape=jax.ShapeDtypeStruct((NP, 256), jnp.float32),
    )(xp, W1)

    hs1, dinv = pl.pallas_call(
        _tc_a1_body,
        grid=grid,
        in_specs=[_row_spec(256), _row_spec(1), _row_spec(1)],
        out_specs=[_half_spec(128), _row_spec(1)],
        out_shape=[jax.ShapeDtypeStruct((2, NP, 128), jnp.float32),
                   jax.ShapeDtypeStruct((NP, 1), jnp.float32)],
    )(h1, d0, d1)

    s1 = prop_l1(hs1.reshape(2 * NP, 128), sd_arr)

    hs2 = pl.pallas_call(
        _tc_c_body,
        grid=grid,
        in_specs=[_half_spec(128), _half_spec(128), _row_spec(1),
                  _whole_spec((2, 128)), _whole_spec((256, 128))],
        out_specs=_half_spec(128),
        out_shape=jax.ShapeDtypeStruct((2, NP, 128), jnp.float32),
    )(s1, hs1, dinv, b1.reshape(2, 128), W2)

    s2 = prop_l2(hs2.reshape(2 * NP, 128), sd_arr)

    vs = pl.pallas_call(
        _tc_e_body,
        grid=grid,
        in_specs=[_half_spec(128), _half_spec(128), _row_spec(1),
                  _whole_spec((1, 128)), _whole_spec((128, 1))],
        out_specs=_row_spec(1),
        out_shape=jax.ShapeDtypeStruct((NP, 1), jnp.float32),
    )(s2, hs2, dinv, b2.reshape(1, 128), W3)

    s3p = prop_l3(vs.reshape(NP), sd_arr)
    s30 = s3p[0].reshape(NP, 1)
    s31 = s3p[1].reshape(NP, 1)

    out = pl.pallas_call(
        _tc_g_body,
        grid=grid,
        in_specs=[_row_spec(1), _row_spec(1), _row_spec(1), _row_spec(1),
                  _whole_spec((1, 1)), _whole_spec((1, 16)),
                  _whole_spec((1, 16))],
        out_specs=_row_spec(16),
        out_shape=jax.ShapeDtypeStruct((NP, 16), jnp.float32),
    )(s30, s31, vs, dinv, b3.reshape(1, 1), fc_W, fc_b.reshape(1, 16))

    return out[:N]


# speedup vs baseline: 1.0097x; 1.0097x over previous
import functools

import jax
import jax.numpy as jnp
from jax import lax
from jax.experimental import pallas as pl
from jax.experimental.pallas import tpu as pltpu
from jax.experimental.pallas import tpu_sc as plsc

N = 10000
E = 160000
NP = 10240
CH = 112
EP = 161280
NSUB = 16
ROWS_PER_SUB = NP // NSUB
GAH = 1
NBUF = GAH + 2
IDXB = GAH + 3

_MESH = plsc.VectorSubcoreMesh(core_axis_name="c", subcore_axis_name="s")


def _make_prop(dh, feat_split, dup_table=False):
    nc = (EP if feat_split else EP // 2) // NSUB // CH
    xform = feat_split or dup_table
    row_shape = (CH, dh) if dh > 1 else (CH,)

    @functools.partial(
        pl.kernel,
        out_type=jax.ShapeDtypeStruct((2, NP, dh) if dh > 1 else (2, NP),
                                      jnp.float32),
        mesh=_MESH,
        scratch_types=(
            [pltpu.VMEM((2, CH), jnp.int32) for _ in range(IDXB)]
            + [pltpu.VMEM(row_shape, jnp.float32) for _ in range(NBUF)]
            + [pltpu.VMEM_SHARED((NP, dh) if dh > 1 else (NP,),
                                 jnp.float32)]
            + [pltpu.SemaphoreType.DMA for _ in range(2 * NBUF + IDXB)]),
    )
    def prop(table, sd_hbm, out, *scr):
        idx_b = scr[:IDXB]
        bufs = scr[IDXB:IDXB + NBUF]
        acc = scr[IDXB + NBUF]
        o = IDXB + NBUF + 1
        gsems = scr[o:o + NBUF]
        ssems = scr[o + NBUF:o + 2 * NBUF]
        isems = scr[o + 2 * NBUF:o + 2 * NBUF + IDXB]
        c = lax.axis_index("c")
        s = lax.axis_index("s")

        z = jnp.zeros((16,), jnp.float32)
        if dh > 1:
            def zrow(i, _):
                for j in range(dh // 16):
                    bufs[0][i, pl.ds(j * 16, 16)] = z
                return 0

            lax.fori_loop(0, CH, zrow, 0, unroll=4)
        else:
            for j in range(CH // 16):
                bufs[0][pl.ds(j * 16, 16)] = z
        base_r = s * ROWS_PER_SUB
        zfull, zrem = divmod(ROWS_PER_SUB, CH)
        for kz in range(zfull):
            pltpu.sync_copy(bufs[0], acc.at[pl.ds(base_r + kz * CH, CH)])
        if zrem:
            pltpu.sync_copy(bufs[0].at[pl.ds(0, zrem)],
                            acc.at[pl.ds(base_r + zfull * CH, zrem)])
        plsc.subcore_barrier()

        off = c * NP

        def fire_idx(k):
            ib = k % IDXB
            if feat_split:
                g = s * nc + k
            else:
                g = (k * 2 + c) * NSUB + s
            return pltpu.async_copy(sd_hbm.at[g], idx_b[ib], isems[ib])

        def prep_gather(k):
            ib = k % IDXB
            idx_d[k].wait()
            if xform:
                for j in range(CH // 16):
                    idx_b[ib][0, pl.ds(j * 16, 16)] = (
                        idx_b[ib][0, pl.ds(j * 16, 16)] + off)
            return pltpu.async_copy(table.at[idx_b[ib].at[0]],
                                    bufs[k % NBUF], gsems[k % NBUF])

        idx_d = [None] * nc
        gdl = [None] * nc
        sd = [None] * nc
        for k in range(min(GAH + 1, nc)):
            idx_d[k] = fire_idx(k)
        for k in range(min(GAH, nc)):
            gdl[k] = prep_gather(k)
        for k in range(nc):
            if k >= 2:
                sd[k - 2].wait()
            if k + GAH + 1 < nc:
                idx_d[k + GAH + 1] = fire_idx(k + GAH + 1)
            if k + GAH < nc:
                gdl[k + GAH] = prep_gather(k + GAH)
            gdl[k].wait()
            sd[k] = pltpu.async_copy(bufs[k % NBUF],
                                     acc.at[idx_b[k % IDXB].at[1]],
                                     ssems[k % NBUF], add=True)
        for k in range(max(0, nc - 2), nc):
            sd[k].wait()
        plsc.subcore_barrier()
        pltpu.sync_copy(
            acc.at[pl.ds(base_r, ROWS_PER_SUB)],
            out.at[c, pl.ds(base_r, ROWS_PER_SUB)],
        )

    return prop


def _make_count():
    nc = EP // 2 // NSUB // CH

    @functools.partial(
        pl.kernel,
        out_type=jax.ShapeDtypeStruct((2, NP), jnp.float32),
        mesh=_MESH,
        scratch_types=[
            pltpu.VMEM((nc, CH), jnp.int32),
            pltpu.VMEM((CH,), jnp.float32),
            pltpu.VMEM_SHARED((NP,), jnp.float32),
            pltpu.SemaphoreType.DMA,
        ],
    )
    def count(dstp, out, dst_loc, ones_v, acc, sem):
        c = lax.axis_index("c")
        s = lax.axis_index("s")
        pltpu.sync_copy(dstp.at[c, s], dst_loc)
        z = jnp.zeros((16,), jnp.float32)
        for j in range(CH // 16):
            ones_v[pl.ds(j * 16, 16)] = z
        base_r = s * ROWS_PER_SUB
        zfull, zrem = divmod(ROWS_PER_SUB, CH)
        for kz in range(zfull):
            pltpu.sync_copy(ones_v, acc.at[pl.ds(base_r + kz * CH, CH)])
        if zrem:
            pltpu.sync_copy(ones_v.at[pl.ds(0, zrem)],
                            acc.at[pl.ds(base_r + zfull * CH, zrem)])
        one = jnp.ones((16,), jnp.float32)
        for j in range(CH // 16):
            ones_v[pl.ds(j * 16, 16)] = one
        plsc.subcore_barrier()
        sds = []
        for k in range(nc):
            sds.append(pltpu.async_copy(ones_v, acc.at[dst_loc.at[k]], sem,
                                        add=True))
        for d in sds:
            d.wait()
        plsc.subcore_barrier()
        pltpu.sync_copy(
            acc.at[pl.ds(base_r, ROWS_PER_SUB)],
            out.at[c, pl.ds(base_r, ROWS_PER_SUB)],
        )

    return count


_R = 512


def _tc_a_body(x_ref, w1_ref, d0_ref, d1_ref, hs_ref, dinv_ref):
    dinv = lax.rsqrt(d0_ref[...] + d1_ref[...] + 1.0)
    h = jnp.dot(x_ref[...], w1_ref[...], preferred_element_type=jnp.float32)
    hs = h * dinv
    hs_ref[0] = hs[:, :128]
    hs_ref[1] = hs[:, 128:]
    dinv_ref[...] = dinv


def _tc_c_body(s_ref, hs_ref, dinv_ref, b1_ref, w2_ref, out_ref):
    dinv = dinv_ref[...]
    b1 = b1_ref[...]
    w2 = w2_ref[...]
    t0 = jax.nn.relu(dinv * (s_ref[0] + hs_ref[0]) + b1[0][None, :])
    t1 = jax.nn.relu(dinv * (s_ref[1] + hs_ref[1]) + b1[1][None, :])
    h2 = (jnp.dot(t0, w2[:128], preferred_element_type=jnp.float32)
          + jnp.dot(t1, w2[128:], preferred_element_type=jnp.float32))
    hs2 = h2 * dinv
    out_ref[0] = hs2
    out_ref[1] = hs2


def _tc_e_body(s_ref, hs_ref, dinv_ref, b2_ref, w3_ref, out_ref):
    dinv = dinv_ref[...]
    t = jax.nn.relu(dinv * (s_ref[0] + s_ref[1] + hs_ref[0]) + b2_ref[...])
    v = jnp.dot(t, w3_ref[...], preferred_element_type=jnp.float32)
    out_ref[...] = v * dinv


def _tc_g_body(s0_ref, s1_ref, vs_ref, dinv_ref, b3_ref, fcw_ref, fcb_ref,
               out_ref):
    h3 = (dinv_ref[...] * (s0_ref[...] + s1_ref[...] + vs_ref[...])
          + b3_ref[0, 0])
    out_ref[...] = h3 * fcw_ref[...] + fcb_ref[...]


def _row_spec(cols):
    return pl.BlockSpec((_R, cols), lambda i: (i, 0))


def _whole_spec(shape):
    return pl.BlockSpec(shape, lambda i: tuple(0 for _ in shape))


def _half_spec(cols):
    return pl.BlockSpec((2, _R, cols), lambda i: (0, i, 0))


def kernel(x, edge_index, W1, b1, W2, b2, W3, b3, fc_W, fc_b):
    grid = (NP // _R,)

    src = edge_index[0].astype(jnp.int32)
    dst = edge_index[1].astype(jnp.int32)
    pad_dst = N + (jnp.arange(EP - E, dtype=jnp.int32) % (NP - N))
    srcp = jnp.concatenate([src, jnp.full((EP - E,), N, jnp.int32)])
    dstp = jnp.concatenate([dst, pad_dst])
    sd_arr = jnp.stack([srcp.reshape(EP // CH, CH),
                        dstp.reshape(EP // CH, CH)], axis=1)
    nc_e = EP // 2 // NSUB // CH
    dst_e = dstp.reshape(2, NSUB, nc_e, CH)

    xp = jnp.pad(x, ((0, NP - N), (0, 0)))

    count1d = _make_count()
    prop_l1 = _make_prop(128, feat_split=True)
    prop_l2 = _make_prop(128, feat_split=False, dup_table=True)
    prop_l3 = _make_prop(1, feat_split=False)

    degp = count1d(dst_e)
    d0 = degp[0].reshape(NP, 1)
    d1 = degp[1].reshape(NP, 1)

    hs1, dinv = pl.pallas_call(
        _tc_a_body,
        grid=grid,
        in_specs=[_row_spec(256), _whole_spec((256, 256)), _row_spec(1),
                  _row_spec(1)],
        out_specs=[_half_spec(128), _row_spec(1)],
        out_shape=[jax.ShapeDtypeStruct((2, NP, 128), jnp.float32),
                   jax.ShapeDtypeStruct((NP, 1), jnp.float32)],
    )(xp, W1, d0, d1)

    s1 = prop_l1(hs1.reshape(2 * NP, 128), sd_arr)

    hs2 = pl.pallas_call(
        _tc_c_body,
        grid=grid,
        in_specs=[_half_spec(128), _half_spec(128), _row_spec(1),
                  _whole_spec((2, 128)), _whole_spec((256, 128))],
        out_specs=_half_spec(128),
        out_shape=jax.ShapeDtypeStruct((2, NP, 128), jnp.float32),
    )(s1, hs1, dinv, b1.reshape(2, 128), W2)

    s2 = prop_l2(hs2.reshape(2 * NP, 128), sd_arr)

    vs = pl.pallas_call(
        _tc_e_body,
        grid=grid,
        in_specs=[_half_spec(128), _half_spec(128), _row_spec(1),
                  _whole_spec((1, 128)), _whole_spec((128, 1))],
        out_specs=_row_spec(1),
        out_shape=jax.ShapeDtypeStruct((NP, 1), jnp.float32),
    )(s2, hs2, dinv, b2.reshape(1, 128), W3)

    s3p = prop_l3(vs.reshape(NP), sd_arr)
    s30 = s3p[0].reshape(NP, 1)
    s31 = s3p[1].reshape(NP, 1)

    out = pl.pallas_call(
        _tc_g_body,
        grid=grid,
        in_specs=[_row_spec(1), _row_spec(1), _row_spec(1), _row_spec(1),
                  _whole_spec((1, 1)), _whole_spec((1, 16)),
                  _whole_spec((1, 16))],
        out_specs=_row_spec(16),
        out_shape=jax.ShapeDtypeStruct((NP, 16), jnp.float32),
    )(s30, s31, vs, dinv, b3.reshape(1, 1), fc_W, fc_b.reshape(1, 16))

    return out[:N]


# speedup vs baseline: 1.0187x; 1.0089x over previous
import functools

import jax
import jax.numpy as jnp
from jax import lax
from jax.experimental import pallas as pl
from jax.experimental.pallas import tpu as pltpu
from jax.experimental.pallas import tpu_sc as plsc

N = 10000
E = 160000
NP = 10240
CH = 112
EP = 161280
NSUB = 16
ROWS_PER_SUB = NP // NSUB
GAH = 1
NBUF = GAH + 2
IDXB = GAH + 3

_MESH = plsc.VectorSubcoreMesh(core_axis_name="c", subcore_axis_name="s")


def _make_prop(dh, feat_split, dup_table=False):
    nc = (EP if feat_split else EP // 2) // NSUB // CH
    xform = feat_split or dup_table
    row_shape = (CH, dh) if dh > 1 else (CH,)

    @functools.partial(
        pl.kernel,
        out_type=jax.ShapeDtypeStruct((2, NP, dh) if dh > 1 else (2, NP),
                                      jnp.float32),
        mesh=_MESH,
        scratch_types=(
            [pltpu.VMEM((CH,), jnp.int32) for _ in range(2 * IDXB)]
            + [pltpu.VMEM(row_shape, jnp.float32) for _ in range(NBUF)]
            + [pltpu.VMEM_SHARED((NP, dh) if dh > 1 else (NP,),
                                 jnp.float32)]
            + [pltpu.SemaphoreType.DMA for _ in range(2 * NBUF + IDXB)]),
    )
    def prop(table, srcp, dstp, out, *scr):
        src_b = scr[:IDXB]
        dst_b = scr[IDXB:2 * IDXB]
        bufs = scr[2 * IDXB:2 * IDXB + NBUF]
        acc = scr[2 * IDXB + NBUF]
        o = 2 * IDXB + NBUF + 1
        gsems = scr[o:o + NBUF]
        ssems = scr[o + NBUF:o + 2 * NBUF]
        isems = scr[o + 2 * NBUF:o + 2 * NBUF + IDXB]
        c = lax.axis_index("c")
        s = lax.axis_index("s")

        z = jnp.zeros((16,), jnp.float32)
        if dh > 1:
            def zrow(i, _):
                for j in range(dh // 16):
                    bufs[0][i, pl.ds(j * 16, 16)] = z
                return 0

            lax.fori_loop(0, CH, zrow, 0, unroll=4)
        else:
            for j in range(CH // 16):
                bufs[0][pl.ds(j * 16, 16)] = z
        base_r = s * ROWS_PER_SUB
        zfull, zrem = divmod(ROWS_PER_SUB, CH)
        for kz in range(zfull):
            pltpu.sync_copy(bufs[0], acc.at[pl.ds(base_r + kz * CH, CH)])
        if zrem:
            pltpu.sync_copy(bufs[0].at[pl.ds(0, zrem)],
                            acc.at[pl.ds(base_r + zfull * CH, zrem)])
        plsc.subcore_barrier()

        off = c * NP

        def fire_idx(k):
            ib = k % IDXB
            if feat_split:
                eo = (s * nc + k) * CH
            else:
                eo = ((k * 2 + c) * NSUB + s) * CH
            di = pltpu.async_copy(srcp.at[pl.ds(eo, CH)], src_b[ib],
                                  isems[ib])
            dj = pltpu.async_copy(dstp.at[pl.ds(eo, CH)], dst_b[ib],
                                  isems[ib])
            return (di, dj)

        def prep_gather(k):
            ib = k % IDXB
            for d in idx_d[k]:
                d.wait()
            if xform:
                for j in range(CH // 16):
                    src_b[ib][pl.ds(j * 16, 16)] = (
                        src_b[ib][pl.ds(j * 16, 16)] + off)
            return pltpu.async_copy(table.at[src_b[ib]], bufs[k % NBUF],
                                    gsems[k % NBUF])

        idx_d = [None] * nc
        gdl = [None] * nc
        sd = [None] * nc
        for k in range(min(GAH + 1, nc)):
            idx_d[k] = fire_idx(k)
        for k in range(min(GAH, nc)):
            gdl[k] = prep_gather(k)
        for k in range(nc):
            if k >= 2:
                sd[k - 2].wait()
            if k + GAH + 1 < nc:
                idx_d[k + GAH + 1] = fire_idx(k + GAH + 1)
            if k + GAH < nc:
                gdl[k + GAH] = prep_gather(k + GAH)
            gdl[k].wait()
            sd[k] = pltpu.async_copy(bufs[k % NBUF], acc.at[dst_b[k % IDXB]],
                                     ssems[k % NBUF], add=True)
        for k in range(max(0, nc - 2), nc):
            sd[k].wait()
        plsc.subcore_barrier()
        pltpu.sync_copy(
            acc.at[pl.ds(base_r, ROWS_PER_SUB)],
            out.at[c, pl.ds(base_r, ROWS_PER_SUB)],
        )

    return prop


def _make_count():
    nc = EP // 2 // NSUB // CH

    @functools.partial(
        pl.kernel,
        out_type=jax.ShapeDtypeStruct((2, NP), jnp.float32),
        mesh=_MESH,
        scratch_types=[
            pltpu.VMEM((nc, CH), jnp.int32),
            pltpu.VMEM((CH,), jnp.float32),
            pltpu.VMEM_SHARED((NP,), jnp.float32),
            pltpu.SemaphoreType.DMA,
        ],
    )
    def count(dstp, out, dst_loc, ones_v, acc, sem):
        c = lax.axis_index("c")
        s = lax.axis_index("s")
        pltpu.sync_copy(dstp.at[c, s], dst_loc)
        z = jnp.zeros((16,), jnp.float32)
        for j in range(CH // 16):
            ones_v[pl.ds(j * 16, 16)] = z
        base_r = s * ROWS_PER_SUB
        zfull, zrem = divmod(ROWS_PER_SUB, CH)
        for kz in range(zfull):
            pltpu.sync_copy(ones_v, acc.at[pl.ds(base_r + kz * CH, CH)])
        if zrem:
            pltpu.sync_copy(ones_v.at[pl.ds(0, zrem)],
                            acc.at[pl.ds(base_r + zfull * CH, zrem)])
        one = jnp.ones((16,), jnp.float32)
        for j in range(CH // 16):
            ones_v[pl.ds(j * 16, 16)] = one
        plsc.subcore_barrier()
        sds = []
        for k in range(nc):
            sds.append(pltpu.async_copy(ones_v, acc.at[dst_loc.at[k]], sem,
                                        add=True))
        for d in sds:
            d.wait()
        plsc.subcore_barrier()
        pltpu.sync_copy(
            acc.at[pl.ds(base_r, ROWS_PER_SUB)],
            out.at[c, pl.ds(base_r, ROWS_PER_SUB)],
        )

    return count


_R = 512


def _tc_a_body(x_ref, w1_ref, d0_ref, d1_ref, hs_ref, dinv_ref):
    dinv = lax.rsqrt(d0_ref[...] + d1_ref[...] + 1.0)
    h = jnp.dot(x_ref[...], w1_ref[...], preferred_element_type=jnp.float32)
    hs = h * dinv
    hs_ref[0] = hs[:, :128]
    hs_ref[1] = hs[:, 128:]
    dinv_ref[...] = dinv


def _tc_c_body(s_ref, hs_ref, dinv_ref, b1_ref, w2_ref, out_ref):
    dinv = dinv_ref[...]
    b1 = b1_ref[...]
    w2 = w2_ref[...]
    t0 = jax.nn.relu(dinv * (s_ref[0] + hs_ref[0]) + b1[0][None, :])
    t1 = jax.nn.relu(dinv * (s_ref[1] + hs_ref[1]) + b1[1][None, :])
    h2 = (jnp.dot(t0, w2[:128], preferred_element_type=jnp.float32)
          + jnp.dot(t1, w2[128:], preferred_element_type=jnp.float32))
    hs2 = h2 * dinv
    out_ref[0] = hs2
    out_ref[1] = hs2


def _tc_e_body(s_ref, hs_ref, dinv_ref, b2_ref, w3_ref, out_ref):
    dinv = dinv_ref[...]
    t = jax.nn.relu(dinv * (s_ref[0] + s_ref[1] + hs_ref[0]) + b2_ref[...])
    v = jnp.dot(t, w3_ref[...], preferred_element_type=jnp.float32)
    out_ref[...] = v * dinv


def _tc_g_body(s0_ref, s1_ref, vs_ref, dinv_ref, b3_ref, fcw_ref, fcb_ref,
               out_ref):
    h3 = (dinv_ref[...] * (s0_ref[...] + s1_ref[...] + vs_ref[...])
          + b3_ref[0, 0])
    out_ref[...] = h3 * fcw_ref[...] + fcb_ref[...]


def _row_spec(cols):
    return pl.BlockSpec((_R, cols), lambda i: (i, 0))


def _whole_spec(shape):
    return pl.BlockSpec(shape, lambda i: tuple(0 for _ in shape))


def _half_spec(cols):
    return pl.BlockSpec((2, _R, cols), lambda i: (0, i, 0))


def kernel(x, edge_index, W1, b1, W2, b2, W3, b3, fc_W, fc_b):
    grid = (NP // _R,)

    src = edge_index[0].astype(jnp.int32)
    dst = edge_index[1].astype(jnp.int32)
    pad_dst = N + (jnp.arange(EP - E, dtype=jnp.int32) % (NP - N))
    srcp = jnp.concatenate([src, jnp.full((EP - E,), N, jnp.int32)])
    dstp = jnp.concatenate([dst, pad_dst])
    nc_e = EP // 2 // NSUB // CH
    dst_e = dstp.reshape(2, NSUB, nc_e, CH)

    xp = jnp.pad(x, ((0, NP - N), (0, 0)))

    count1d = _make_count()
    prop_l1 = _make_prop(128, feat_split=True)
    prop_l2 = _make_prop(128, feat_split=False, dup_table=True)
    prop_l3 = _make_prop(1, feat_split=False)

    degp = count1d(dst_e)
    d0 = degp[0].reshape(NP, 1)
    d1 = degp[1].reshape(NP, 1)

    hs1, dinv = pl.pallas_call(
        _tc_a_body,
        grid=grid,
        in_specs=[_row_spec(256), _whole_spec((256, 256)), _row_spec(1),
                  _row_spec(1)],
        out_specs=[_half_spec(128), _row_spec(1)],
        out_shape=[jax.ShapeDtypeStruct((2, NP, 128), jnp.float32),
                   jax.ShapeDtypeStruct((NP, 1), jnp.float32)],
    )(xp, W1, d0, d1)

    s1 = prop_l1(hs1.reshape(2 * NP, 128), srcp, dstp)

    hs2 = pl.pallas_call(
        _tc_c_body,
        grid=grid,
        in_specs=[_half_spec(128), _half_spec(128), _row_spec(1),
                  _whole_spec((2, 128)), _whole_spec((256, 128))],
        out_specs=_half_spec(128),
        out_shape=jax.ShapeDtypeStruct((2, NP, 128), jnp.float32),
    )(s1, hs1, dinv, b1.reshape(2, 128), W2)

    s2 = prop_l2(hs2.reshape(2 * NP, 128), srcp, dstp)

    vs = pl.pallas_call(
        _tc_e_body,
        grid=grid,
        in_specs=[_half_spec(128), _half_spec(128), _row_spec(1),
                  _whole_spec((1, 128)), _whole_spec((128, 1))],
        out_specs=_row_spec(1),
        out_shape=jax.ShapeDtypeStruct((NP, 1), jnp.float32),
    )(s2, hs2, dinv, b2.reshape(1, 128), W3)

    s3p = prop_l3(vs.reshape(NP), srcp, dstp)
    s30 = s3p[0].reshape(NP, 1)
    s31 = s3p[1].reshape(NP, 1)

    out = pl.pallas_call(
        _tc_g_body,
        grid=grid,
        in_specs=[_row_spec(1), _row_spec(1), _row_spec(1), _row_spec(1),
                  _whole_spec((1, 1)), _whole_spec((1, 16)),
                  _whole_spec((1, 16))],
        out_specs=_row_spec(16),
        out_shape=jax.ShapeDtypeStruct((NP, 16), jnp.float32),
    )(s30, s31, vs, dinv, b3.reshape(1, 1), fc_W, fc_b.reshape(1, 16))

    return out[:N]
